# Initial kernel scaffold; baseline (speedup 1.0000x reference)
import jax
import jax.numpy as jnp
from jax.experimental import pallas as pl


def kernel(entity, edge_index, edge_type, edge_norm, emb, basis1, att1, root1, bias1, basis2, att2, root2, bias2):
    raise NotImplementedError("write your pallas kernel here")


# speedup vs baseline: 3.8375x; 3.8375x over previous
import jax
import jax.numpy as jnp
from jax import lax
from jax.experimental import pallas as pl
from jax.experimental.pallas import tpu as pltpu
from jax.experimental.pallas import tpu_sc as plsc

NC = 2
NS = 16
NW = NC * NS
L = 16
C = 32
GPC = C // L
ROWW = 128


def _sc_edge_pass(nrows, ept, d, b, y, meta, attf):
    bd = b * d
    nch = ept // C
    zr = nrows // NS
    assert nch % 2 == 0 and nch >= 4 and zr % 8 == 0

    mesh = plsc.VectorSubcoreMesh(core_axis_name="c", subcore_axis_name="s")

    def body(y_hbm, meta_hbm, att_hbm, p_hbm,
             mc, att_v, srcc0, srcc1, dstc0, dstc1, wv, rows0, rows1, msg,
             shared, gsem0, gsem1, msem0, msem1):
        srccs = (srcc0, srcc1)
        dstcs = (dstc0, dstc1)
        rowss = (rows0, rows1)
        gsems = (gsem0, gsem1)
        msems = (msem0, msem1)
        cid = lax.axis_index("c")
        sid = lax.axis_index("s")
        wid = cid * NS + sid
        wbase = wid * ept * 4

        pltpu.sync_copy(att_hbm, att_v)

        zero16 = jnp.zeros((L,), jnp.float32)
        iota16 = lax.iota(jnp.int32, L)
        cvec = jnp.where(iota16 == 0, 1.0, 0.0).astype(jnp.float32)

        def _zr(r, carry):
            for k in range(ROWW // L):
                msg[r, pl.ds(k * L, L)] = zero16
            return carry
        lax.fori_loop(0, C, _zr, 0)
        zbase = sid * zr
        nfull = zr // C
        for j in range(nfull):
            pltpu.sync_copy(msg.at[pl.ds(0, C)],
                            shared.at[pl.ds(zbase + j * C, C)])
        rem = zr - nfull * C
        if rem:
            pltpu.sync_copy(msg.at[pl.ds(0, rem)],
                            shared.at[pl.ds(zbase + nfull * C, rem)])
        plsc.subcore_barrier()

        def start_meta(i, bb):
            pltpu.async_copy(meta_hbm.at[pl.ds(wbase + i * C * 4, C * 4)],
                             mc.at[bb], msems[bb])

        def wait_meta(i, bb):
            pltpu.make_async_copy(meta_hbm.at[pl.ds(wbase + i * C * 4, C * 4)],
                                  mc.at[bb], msems[bb]).wait()

        def fill_idx(bb):
            for g in range(GPC):
                srccs[bb][pl.ds(g * L, L)] = mc[bb, pl.ds(g * 64, L)]
                dstcs[bb][pl.ds(g * L, L)] = mc[bb, pl.ds(g * 64 + L, L)]

        def start_gather(bb):
            pltpu.async_copy(y_hbm.at[srccs[bb]], rowss[bb], gsems[bb])

        def wait_gather(bb):
            pltpu.make_async_copy(y_hbm.at[srccs[bb]], rowss[bb],
                                  gsems[bb]).wait()

        def compute_w(bb):
            for g in range(GPC):
                t = mc[bb, pl.ds(g * 64 + 2 * L, L)]
                en = plsc.bitcast(mc[bb, pl.ds(g * 64 + 3 * L, L)],
                                  jnp.float32)
                tb = t * b
                for b8 in range(b):
                    co = plsc.load_gather(att_v, [tb + b8])
                    wv[pl.ds(b8 * C + g * L, L)] = co * en

        def compute_msg(bb):
            rows = rowss[bb]

            def ebody(e, carry):
                accs = None
                for b8 in range(b):
                    wb = plsc.load_gather(
                        wv, [jnp.full((L,), b8 * C, jnp.int32) + e])
                    cur = [wb * rows[e, pl.ds(b8 * d + k * L, L)]
                           for k in range(d // L)]
                    if accs is None:
                        accs = cur
                    else:
                        accs = [a + c2 for a, c2 in zip(accs, cur)]
                for k in range(d // L):
                    msg[e, pl.ds(k * L, L)] = accs[k]
                msg[e, pl.ds(d, L)] = cvec
                return carry
            lax.fori_loop(0, C, ebody, 0)

        def scatter(bb):
            pltpu.sync_copy(msg, shared.at[dstcs[bb]], add=True)

        pltpu.sync_copy(meta_hbm.at[pl.ds(wbase, C * 4)], mc.at[0])
        fill_idx(0)
        start_gather(0)
        start_meta(1, 1)

        def phase_full(i, bb):
            wait_meta(i + 1, bb ^ 1)
            fill_idx(bb ^ 1)
            start_gather(bb ^ 1)
            wait_gather(bb)
            compute_w(bb)
            start_meta(i + 2, bb)
            compute_msg(bb)
            scatter(bb)

        def outer(i2, carry):
            phase_full(2 * i2, 0)
            phase_full(2 * i2 + 1, 1)
            return carry
        lax.fori_loop(0, nch // 2 - 1, outer, 0)

        wait_meta(nch - 1, 1)
        fill_idx(1)
        start_gather(1)
        wait_gather(0)
        compute_w(0)
        compute_msg(0)
        scatter(0)
        wait_gather(1)
        compute_w(1)
        compute_msg(1)
        scatter(1)

        plsc.subcore_barrier()
        pltpu.sync_copy(shared.at[pl.ds(zbase, zr)],
                        p_hbm.at[cid, pl.ds(zbase, zr)])

    rb = attf.shape[0]
    run = pl.kernel(
        body,
        out_type=jax.ShapeDtypeStruct((NC, nrows, ROWW), jnp.float32),
        mesh=mesh,
        compiler_params=pltpu.CompilerParams(needs_layout_passes=False),
        scratch_types=[
            pltpu.VMEM((2, C * 4), jnp.int32),
            pltpu.VMEM((rb,), jnp.float32),
            pltpu.VMEM((C,), jnp.int32),
            pltpu.VMEM((C,), jnp.int32),
            pltpu.VMEM((C,), jnp.int32),
            pltpu.VMEM((C,), jnp.int32),
            pltpu.VMEM((b * C,), jnp.float32),
            pltpu.VMEM((C, bd), jnp.float32),
            pltpu.VMEM((C, bd), jnp.float32),
            pltpu.VMEM((C, ROWW), jnp.float32),
            pltpu.VMEM_SHARED((nrows, ROWW), jnp.float32),
            pltpu.SemaphoreType.DMA,
            pltpu.SemaphoreType.DMA,
            pltpu.SemaphoreType.DMA,
            pltpu.SemaphoreType.DMA,
        ],
    )
    return run(y, meta, attf)


def _dense_pre(x, wcat, root, blk):
    n, d = x.shape
    bd = wcat.shape[1]

    def body(x_ref, wcat_ref, root_ref, y_ref, xr_ref):
        xb = x_ref[...]
        y_ref[...] = jnp.dot(xb, wcat_ref[...],
                             preferred_element_type=jnp.float32)
        xr_ref[...] = jnp.dot(xb, root_ref[...],
                              preferred_element_type=jnp.float32)

    return pl.pallas_call(
        body,
        grid=(n // blk,),
        in_specs=[
            pl.BlockSpec((blk, d), lambda i: (i, 0)),
            pl.BlockSpec((d, bd), lambda i: (0, 0)),
            pl.BlockSpec((d, d), lambda i: (0, 0)),
        ],
        out_specs=[
            pl.BlockSpec((blk, bd), lambda i: (i, 0)),
            pl.BlockSpec((blk, d), lambda i: (i, 0)),
        ],
        out_shape=[
            jax.ShapeDtypeStruct((n, bd), jnp.float32),
            jax.ShapeDtypeStruct((n, d), jnp.float32),
        ],
    )(x, wcat, root)


def _combine(p_ref, xr_ref, bias_ref, d):
    p = p_ref[...]
    s = p[0, :, :d] + p[1, :, :d]
    c = p[0, :, d:d + 1] + p[1, :, d:d + 1]
    return s / jnp.maximum(c, 1.0) + xr_ref[...] + bias_ref[...]


def _dense_mid(p, xr, bias, wcat, root, blk):
    n, d = xr.shape
    bd = wcat.shape[1]

    def body(p_ref, xr_ref, bias_ref, wcat_ref, root_ref, y_ref, xr2_ref):
        x1 = _combine(p_ref, xr_ref, bias_ref, d)
        y_ref[...] = jnp.dot(x1, wcat_ref[...],
                             preferred_element_type=jnp.float32)
        xr2_ref[...] = jnp.dot(x1, root_ref[...],
                               preferred_element_type=jnp.float32)

    return pl.pallas_call(
        body,
        grid=(n // blk,),
        in_specs=[
            pl.BlockSpec((NC, blk, ROWW), lambda i: (0, i, 0)),
            pl.BlockSpec((blk, d), lambda i: (i, 0)),
            pl.BlockSpec((1, d), lambda i: (0, 0)),
            pl.BlockSpec((d, bd), lambda i: (0, 0)),
            pl.BlockSpec((d, d), lambda i: (0, 0)),
        ],
        out_specs=[
            pl.BlockSpec((blk, bd), lambda i: (i, 0)),
            pl.BlockSpec((blk, d), lambda i: (i, 0)),
        ],
        out_shape=[
            jax.ShapeDtypeStruct((n, bd), jnp.float32),
            jax.ShapeDtypeStruct((n, d), jnp.float32),
        ],
    )(p, xr, bias, wcat, root)


def _dense_post(p, xr, bias, blk):
    n, d = xr.shape

    def body(p_ref, xr_ref, bias_ref, out_ref):
        out_ref[...] = _combine(p_ref, xr_ref, bias_ref, d)

    return pl.pallas_call(
        body,
        grid=(n // blk,),
        in_specs=[
            pl.BlockSpec((NC, blk, ROWW), lambda i: (0, i, 0)),
            pl.BlockSpec((blk, d), lambda i: (i, 0)),
            pl.BlockSpec((1, d), lambda i: (0, 0)),
        ],
        out_specs=pl.BlockSpec((blk, d), lambda i: (i, 0)),
        out_shape=jax.ShapeDtypeStruct((n, d), jnp.float32),
    )(p, xr, bias)


def kernel(entity, edge_index, edge_type, edge_norm, emb,
           basis1, att1, root1, bias1, basis2, att2, root2, bias2):
    n, d = emb.shape
    b = basis1.shape[0]
    e = edge_type.shape[0]

    x0 = jnp.take(emb, entity, axis=0)
    src = edge_index[0]
    dst = edge_index[1]

    ept = -(-e // (NW * 2 * C)) * (2 * C)
    pad = NW * ept - e
    srcp = jnp.pad(src, (0, pad))
    dstp = jnp.pad(dst, (0, pad), constant_values=n)
    etp = jnp.pad(edge_type, (0, pad))
    enp = jnp.pad(edge_norm, (0, pad))
    meta = jnp.stack([
        srcp.reshape(-1, L), dstp.reshape(-1, L), etp.reshape(-1, L),
        enp.view(jnp.int32).reshape(-1, L)], axis=1).reshape(-1)
    nrows = -(-(n + 1) // (NS * 8)) * (NS * 8)

    wcat1 = basis1.transpose(1, 0, 2).reshape(d, b * d)
    wcat2 = basis2.transpose(1, 0, 2).reshape(d, b * d)
    attf1 = att1.reshape(-1)
    attf2 = att2.reshape(-1)
    blk = 400

    y1, xr1 = _dense_pre(x0, wcat1, root1, blk)
    p1 = _sc_edge_pass(nrows, ept, d, b, y1, meta, attf1)
    y2, xr2 = _dense_mid(p1, xr1, bias1.reshape(1, -1), wcat2, root2, blk)
    p2 = _sc_edge_pass(nrows, ept, d, b, y2, meta, attf2)
    return _dense_post(p2, xr2, bias2.reshape(1, -1), blk)


# speedup vs baseline: 3.9475x; 1.0287x over previous
import jax
import jax.numpy as jnp
from jax import lax
from jax.experimental import pallas as pl
from jax.experimental.pallas import tpu as pltpu
from jax.experimental.pallas import tpu_sc as plsc

NC = 2
NS = 16
NW = NC * NS
L = 16
C = 32
GPC = C // L
ROWW = 128


def _sc_edge_pass(nrows, ept, d, b, y, meta, attf):
    bd = b * d
    nch = ept // C
    zr = nrows // NS
    assert nch % 2 == 0 and nch >= 4 and zr % 8 == 0

    mesh = plsc.VectorSubcoreMesh(core_axis_name="c", subcore_axis_name="s")

    def body(y_hbm, meta_hbm, att_hbm, p_hbm,
             mc, att_v, srcc0, srcc1, dstc0, dstc1, wv, rows0, rows1, msg,
             shared, gsem0, gsem1, msem0, msem1):
        srccs = (srcc0, srcc1)
        dstcs = (dstc0, dstc1)
        rowss = (rows0, rows1)
        gsems = (gsem0, gsem1)
        msems = (msem0, msem1)
        cid = lax.axis_index("c")
        sid = lax.axis_index("s")
        wid = cid * NS + sid
        wbase = wid * ept * 4

        pltpu.sync_copy(att_hbm, att_v)

        zero16 = jnp.zeros((L,), jnp.float32)
        iota16 = lax.iota(jnp.int32, L)
        cvec = jnp.where(iota16 == 0, 1.0, 0.0).astype(jnp.float32)

        def _zr(r, carry):
            for k in range(ROWW // L):
                msg[r, pl.ds(k * L, L)] = zero16
            return carry
        lax.fori_loop(0, C, _zr, 0)
        zbase = sid * zr
        nfull = zr // C
        for j in range(nfull):
            pltpu.sync_copy(msg.at[pl.ds(0, C)],
                            shared.at[pl.ds(zbase + j * C, C)])
        rem = zr - nfull * C
        if rem:
            pltpu.sync_copy(msg.at[pl.ds(0, rem)],
                            shared.at[pl.ds(zbase + nfull * C, rem)])
        plsc.subcore_barrier()

        def start_meta(i, bb):
            pltpu.async_copy(meta_hbm.at[pl.ds(wbase + i * C * 4, C * 4)],
                             mc.at[bb], msems[bb])

        def wait_meta(i, bb):
            pltpu.make_async_copy(meta_hbm.at[pl.ds(wbase + i * C * 4, C * 4)],
                                  mc.at[bb], msems[bb]).wait()

        def fill_idx(bb):
            for g in range(GPC):
                srccs[bb][pl.ds(g * L, L)] = mc[bb, pl.ds(g * 64, L)]
                dstcs[bb][pl.ds(g * L, L)] = mc[bb, pl.ds(g * 64 + L, L)]

        def start_gather(bb):
            pltpu.async_copy(y_hbm.at[srccs[bb]], rowss[bb], gsems[bb])

        def wait_gather(bb):
            pltpu.make_async_copy(y_hbm.at[srccs[bb]], rowss[bb],
                                  gsems[bb]).wait()

        def compute_w(bb):
            for g in range(GPC):
                t = mc[bb, pl.ds(g * 64 + 2 * L, L)]
                en = plsc.bitcast(mc[bb, pl.ds(g * 64 + 3 * L, L)],
                                  jnp.float32)
                tb = t * b
                for b8 in range(b):
                    co = plsc.load_gather(att_v, [tb + b8])
                    wv[pl.ds(b8 * C + g * L, L)] = co * en

        def compute_msg(bb):
            rows = rowss[bb]

            @plsc.parallel_loop(0, C, unroll=2)
            def ebody(e):
                accs = None
                for b8 in range(b):
                    wb = plsc.load_gather(
                        wv, [jnp.full((L,), b8 * C, jnp.int32) + e])
                    cur = [wb * rows[e, pl.ds(b8 * d + k * L, L)]
                           for k in range(d // L)]
                    if accs is None:
                        accs = cur
                    else:
                        accs = [a + c2 for a, c2 in zip(accs, cur)]
                for k in range(d // L):
                    msg[e, pl.ds(k * L, L)] = accs[k]
                msg[e, pl.ds(d, L)] = cvec

        def scatter(bb):
            pltpu.sync_copy(msg, shared.at[dstcs[bb]], add=True)

        pltpu.sync_copy(meta_hbm.at[pl.ds(wbase, C * 4)], mc.at[0])
        fill_idx(0)
        start_gather(0)
        start_meta(1, 1)

        def phase_full(i, bb):
            wait_meta(i + 1, bb ^ 1)
            fill_idx(bb ^ 1)
            start_gather(bb ^ 1)
            wait_gather(bb)
            compute_w(bb)
            start_meta(i + 2, bb)
            compute_msg(bb)
            scatter(bb)

        def outer(i2, carry):
            phase_full(2 * i2, 0)
            phase_full(2 * i2 + 1, 1)
            return carry
        lax.fori_loop(0, nch // 2 - 1, outer, 0)

        wait_meta(nch - 1, 1)
        fill_idx(1)
        start_gather(1)
        wait_gather(0)
        compute_w(0)
        compute_msg(0)
        scatter(0)
        wait_gather(1)
        compute_w(1)
        compute_msg(1)
        scatter(1)

        plsc.subcore_barrier()
        pltpu.sync_copy(shared.at[pl.ds(zbase, zr)],
                        p_hbm.at[cid, pl.ds(zbase, zr)])

    rb = attf.shape[0]
    run = pl.kernel(
        body,
        out_type=jax.ShapeDtypeStruct((NC, nrows, ROWW), jnp.float32),
        mesh=mesh,
        compiler_params=pltpu.CompilerParams(needs_layout_passes=False),
        scratch_types=[
            pltpu.VMEM((2, C * 4), jnp.int32),
            pltpu.VMEM((rb,), jnp.float32),
            pltpu.VMEM((C,), jnp.int32),
            pltpu.VMEM((C,), jnp.int32),
            pltpu.VMEM((C,), jnp.int32),
            pltpu.VMEM((C,), jnp.int32),
            pltpu.VMEM((b * C,), jnp.float32),
            pltpu.VMEM((C, bd), jnp.float32),
            pltpu.VMEM((C, bd), jnp.float32),
            pltpu.VMEM((C, ROWW), jnp.float32),
            pltpu.VMEM_SHARED((nrows, ROWW), jnp.float32),
            pltpu.SemaphoreType.DMA,
            pltpu.SemaphoreType.DMA,
            pltpu.SemaphoreType.DMA,
            pltpu.SemaphoreType.DMA,
        ],
    )
    return run(y, meta, attf)


def _dense_pre(x, wcat, root, blk):
    n, d = x.shape
    bd = wcat.shape[1]

    def body(x_ref, wcat_ref, root_ref, y_ref, xr_ref):
        xb = x_ref[...]
        y_ref[...] = jnp.dot(xb, wcat_ref[...],
                             preferred_element_type=jnp.float32)
        xr_ref[...] = jnp.dot(xb, root_ref[...],
                              preferred_element_type=jnp.float32)

    return pl.pallas_call(
        body,
        grid=(n // blk,),
        in_specs=[
            pl.BlockSpec((blk, d), lambda i: (i, 0)),
            pl.BlockSpec((d, bd), lambda i: (0, 0)),
            pl.BlockSpec((d, d), lambda i: (0, 0)),
        ],
        out_specs=[
            pl.BlockSpec((blk, bd), lambda i: (i, 0)),
            pl.BlockSpec((blk, d), lambda i: (i, 0)),
        ],
        out_shape=[
            jax.ShapeDtypeStruct((n, bd), jnp.float32),
            jax.ShapeDtypeStruct((n, d), jnp.float32),
        ],
    )(x, wcat, root)


def _combine(p_ref, xr_ref, bias_ref, d):
    p = p_ref[...]
    s = p[0, :, :d] + p[1, :, :d]
    c = p[0, :, d:d + 1] + p[1, :, d:d + 1]
    return s / jnp.maximum(c, 1.0) + xr_ref[...] + bias_ref[...]


def _dense_mid(p, xr, bias, wcat, root, blk):
    n, d = xr.shape
    bd = wcat.shape[1]

    def body(p_ref, xr_ref, bias_ref, wcat_ref, root_ref, y_ref, xr2_ref):
        x1 = _combine(p_ref, xr_ref, bias_ref, d)
        y_ref[...] = jnp.dot(x1, wcat_ref[...],
                             preferred_element_type=jnp.float32)
        xr2_ref[...] = jnp.dot(x1, root_ref[...],
                               preferred_element_type=jnp.float32)

    return pl.pallas_call(
        body,
        grid=(n // blk,),
        in_specs=[
            pl.BlockSpec((NC, blk, ROWW), lambda i: (0, i, 0)),
            pl.BlockSpec((blk, d), lambda i: (i, 0)),
            pl.BlockSpec((1, d), lambda i: (0, 0)),
            pl.BlockSpec((d, bd), lambda i: (0, 0)),
            pl.BlockSpec((d, d), lambda i: (0, 0)),
        ],
        out_specs=[
            pl.BlockSpec((blk, bd), lambda i: (i, 0)),
            pl.BlockSpec((blk, d), lambda i: (i, 0)),
        ],
        out_shape=[
            jax.ShapeDtypeStruct((n, bd), jnp.float32),
            jax.ShapeDtypeStruct((n, d), jnp.float32),
        ],
    )(p, xr, bias, wcat, root)


def _dense_post(p, xr, bias, blk):
    n, d = xr.shape

    def body(p_ref, xr_ref, bias_ref, out_ref):
        out_ref[...] = _combine(p_ref, xr_ref, bias_ref, d)

    return pl.pallas_call(
        body,
        grid=(n // blk,),
        in_specs=[
            pl.BlockSpec((NC, blk, ROWW), lambda i: (0, i, 0)),
            pl.BlockSpec((blk, d), lambda i: (i, 0)),
            pl.BlockSpec((1, d), lambda i: (0, 0)),
        ],
        out_specs=pl.BlockSpec((blk, d), lambda i: (i, 0)),
        out_shape=jax.ShapeDtypeStruct((n, d), jnp.float32),
    )(p, xr, bias)


def kernel(entity, edge_index, edge_type, edge_norm, emb,
           basis1, att1, root1, bias1, basis2, att2, root2, bias2):
    n, d = emb.shape
    b = basis1.shape[0]
    e = edge_type.shape[0]

    x0 = jnp.take(emb, entity, axis=0)
    src = edge_index[0]
    dst = edge_index[1]

    ept = -(-e // (NW * 2 * C)) * (2 * C)
    pad = NW * ept - e
    srcp = jnp.pad(src, (0, pad))
    dstp = jnp.pad(dst, (0, pad), constant_values=n)
    etp = jnp.pad(edge_type, (0, pad))
    enp = jnp.pad(edge_norm, (0, pad))
    meta = jnp.stack([
        srcp.reshape(-1, L), dstp.reshape(-1, L), etp.reshape(-1, L),
        enp.view(jnp.int32).reshape(-1, L)], axis=1).reshape(-1)
    nrows = -(-(n + 1) // (NS * 8)) * (NS * 8)

    wcat1 = basis1.transpose(1, 0, 2).reshape(d, b * d)
    wcat2 = basis2.transpose(1, 0, 2).reshape(d, b * d)
    attf1 = att1.reshape(-1)
    attf2 = att2.reshape(-1)
    blk = 400

    y1, xr1 = _dense_pre(x0, wcat1, root1, blk)
    p1 = _sc_edge_pass(nrows, ept, d, b, y1, meta, attf1)
    y2, xr2 = _dense_mid(p1, xr1, bias1.reshape(1, -1), wcat2, root2, blk)
    p2 = _sc_edge_pass(nrows, ept, d, b, y2, meta, attf2)
    return _dense_post(p2, xr2, bias2.reshape(1, -1), blk)
